# baseline (speedup 1.0000x reference)
import jax
import jax.numpy as jnp
from jax import lax
from jax.experimental import pallas as pl
from jax.experimental.pallas import tpu as pltpu
from jax.experimental.pallas import tpu_sc as plsc

NB = 16
C = 768
HW = 32 * 32
N = NB * HW
NC = 2
NS = 16
NW = NC * NS
ROWS_PER_W = N // NW
PCHUNK = 32
NCHUNK = ROWS_PER_W // PCHUNK
CG = C // 16


def _shuffle_body(x_hbm, idx_hbm, out_hbm, idx_v, in0, in1, out0, out1,
                  gsem0, gsem1, ssem0, ssem1):
    wid = lax.axis_index("s") * NC + lax.axis_index("c")
    base = wid * ROWS_PER_W

    pltpu.sync_copy(idx_hbm, idx_v)

    ins = (in0, in1)
    outs = (out0, out1)
    gsems = (gsem0, gsem1)
    ssems = (ssem0, ssem1)

    def gather(k):
        s = k % 2
        return pltpu.async_copy(
            x_hbm.at[pl.ds(base + k * PCHUNK, PCHUNK)], ins[s], gsems[s])

    def scatter(k):
        s = k % 2
        return pltpu.async_copy(
            outs[s], out_hbm.at[pl.ds(base + k * PCHUNK, PCHUNK)], ssems[s])

    def permute_chunk(in_buf, out_buf):
        @plsc.parallel_loop(0, CG)
        def body(j):
            csl = pl.ds(j * 16, 16)
            idxv = idx_v[csl]
            for p0 in range(0, PCHUNK, 16):
                gs = [
                    plsc.load_gather(
                        in_buf,
                        [jnp.full((16,), p0 + i, dtype=jnp.int32), idxv])
                    for i in range(16)
                ]
                for i in range(16):
                    out_buf[p0 + i, csl] = gs[i]

    gathers = [None, None]
    scatters = [None, None]
    gathers[0] = gather(0)
    for k in range(NCHUNK):
        s = k % 2
        gathers[s].wait()
        if k + 1 < NCHUNK:
            gathers[1 - s] = gather(k + 1)
        if scatters[s] is not None:
            scatters[s].wait()
        permute_chunk(ins[s], outs[s])
        scatters[s] = scatter(k)
    scatters[0].wait()
    scatters[1].wait()


@jax.jit
def _shuffle(xt, indices):
    mesh = plsc.VectorSubcoreMesh(core_axis_name="c", subcore_axis_name="s")
    return pl.kernel(
        _shuffle_body,
        out_type=jax.ShapeDtypeStruct((N, C), jnp.float32),
        mesh=mesh,
        scratch_types=[
            pltpu.VMEM((C,), jnp.int32),
            pltpu.VMEM((PCHUNK, C), jnp.float32),
            pltpu.VMEM((PCHUNK, C), jnp.float32),
            pltpu.VMEM((PCHUNK, C), jnp.float32),
            pltpu.VMEM((PCHUNK, C), jnp.float32),
            pltpu.SemaphoreType.DMA,
            pltpu.SemaphoreType.DMA,
            pltpu.SemaphoreType.DMA,
            pltpu.SemaphoreType.DMA,
        ],
        compiler_params=pltpu.CompilerParams(needs_layout_passes=False),
    )(xt, indices)


def kernel(x, indices):
    xt = jnp.transpose(x, (0, 2, 3, 1)).reshape(N, C)
    yt = _shuffle(xt, indices)
    y = jnp.transpose(yt.reshape(NB, 32, 32, C), (0, 3, 1, 2))
    return (y, jnp.zeros((), dtype=x.dtype))


# speedup vs baseline: 1.0463x; 1.0463x over previous
import jax
import jax.numpy as jnp
from jax import lax
from jax.experimental import pallas as pl
from jax.experimental.pallas import tpu as pltpu
from jax.experimental.pallas import tpu_sc as plsc

NB = 16
C = 768
HW = 32 * 32
N = NB * HW
NC = 2
NS = 16
NW = NC * NS
ROWS_PER_W = N // NW
PCHUNK = 32
NCHUNK = ROWS_PER_W // PCHUNK
CG = C // 16


def _shuffle_body(x_hbm, idx_hbm, out_hbm, idx_v, in0, in1, in2, out0, out1,
                  gsem0, gsem1, gsem2, ssem0, ssem1):
    wid = lax.axis_index("s") * NC + lax.axis_index("c")
    base = wid * ROWS_PER_W

    pltpu.sync_copy(idx_hbm, idx_v)

    ins = (in0, in1, in2)
    outs = (out0, out1)
    gsems = (gsem0, gsem1, gsem2)
    ssems = (ssem0, ssem1)

    def gather(k):
        s = k % 3
        return pltpu.async_copy(
            x_hbm.at[pl.ds(base + k * PCHUNK, PCHUNK)], ins[s], gsems[s])

    def scatter(k):
        s = k % 2
        return pltpu.async_copy(
            outs[s], out_hbm.at[pl.ds(base + k * PCHUNK, PCHUNK)], ssems[s])

    def permute_chunk(in_buf, out_buf):
        @plsc.parallel_loop(0, CG)
        def body(j):
            csl = pl.ds(j * 16, 16)
            idxv = idx_v[csl]
            for p0 in range(0, PCHUNK, 8):
                gs = [
                    plsc.load_gather(
                        in_buf,
                        [jnp.full((16,), p0 + i, dtype=jnp.int32), idxv])
                    for i in range(8)
                ]
                for i in range(8):
                    out_buf[p0 + i, csl] = gs[i]

    gathers = [None, None, None]
    scatters = [None, None]
    gathers[0] = gather(0)
    gathers[1] = gather(1)
    for k in range(NCHUNK):
        gs = k % 3
        ss = k % 2
        gathers[gs].wait()
        if k + 2 < NCHUNK:
            gathers[(k + 2) % 3] = gather(k + 2)
        if scatters[ss] is not None:
            scatters[ss].wait()
        permute_chunk(ins[gs], outs[ss])
        scatters[ss] = scatter(k)
    scatters[0].wait()
    scatters[1].wait()


@jax.jit
def _shuffle(xt, indices):
    mesh = plsc.VectorSubcoreMesh(core_axis_name="c", subcore_axis_name="s")
    return pl.kernel(
        _shuffle_body,
        out_type=jax.ShapeDtypeStruct((N, C), jnp.float32),
        mesh=mesh,
        scratch_types=[
            pltpu.VMEM((C,), jnp.int32),
            pltpu.VMEM((PCHUNK, C), jnp.float32),
            pltpu.VMEM((PCHUNK, C), jnp.float32),
            pltpu.VMEM((PCHUNK, C), jnp.float32),
            pltpu.VMEM((PCHUNK, C), jnp.float32),
            pltpu.VMEM((PCHUNK, C), jnp.float32),
            pltpu.SemaphoreType.DMA,
            pltpu.SemaphoreType.DMA,
            pltpu.SemaphoreType.DMA,
            pltpu.SemaphoreType.DMA,
            pltpu.SemaphoreType.DMA,
        ],
        compiler_params=pltpu.CompilerParams(needs_layout_passes=False),
    )(xt, indices)


def kernel(x, indices):
    xt = jnp.transpose(x, (0, 2, 3, 1)).reshape(N, C)
    yt = _shuffle(xt, indices)
    y = jnp.transpose(yt.reshape(NB, 32, 32, C), (0, 3, 1, 2))
    return (y, jnp.zeros((), dtype=x.dtype))
